# Initial kernel scaffold; baseline (speedup 1.0000x reference)
import jax
import jax.numpy as jnp
from jax.experimental import pallas as pl


def kernel(x, edge_index, W0, b0, W1, b1):
    raise NotImplementedError("write your pallas kernel here")


# speedup vs baseline: 34.3132x; 34.3132x over previous
import functools

import jax
import jax.numpy as jnp
from jax import lax
from jax.experimental import pallas as pl
from jax.experimental.pallas import tpu as pltpu
from jax.experimental.pallas import tpu_sc as plsc

N = 10000
E = 320000
D = 128
H = 16

NC = 2
NS = 16
L = 16
NW = NC * NS
EPW = E // NW
RPT = N // NS
CH = 80

_mesh = plsc.VectorSubcoreMesh(
    core_axis_name="c", subcore_axis_name="s", num_cores=NC, num_subcores=NS
)
_sc_params = pltpu.CompilerParams(
    use_tc_tiling_on_sc=False, needs_layout_passes=False
)


def _worker_id():
    return lax.axis_index("s") * NC + lax.axis_index("c")


@functools.partial(
    pl.kernel,
    out_type=jax.ShapeDtypeStruct((2, NW, N), jnp.float32),
    mesh=_mesh,
    compiler_params=_sc_params,
    scratch_types=[
        pltpu.VMEM((EPW,), jnp.int32),
        pltpu.VMEM((EPW,), jnp.int32),
        pltpu.VMEM((N,), jnp.float32),
        pltpu.VMEM((N,), jnp.float32),
    ],
)
def _sc_degrees(ei_hbm, out_hbm, src_v, dst_v, hist_s, hist_d):
    w = _worker_id()
    base = w * EPW
    pltpu.sync_copy(ei_hbm.at[0, pl.ds(base, EPW)], src_v)
    pltpu.sync_copy(ei_hbm.at[1, pl.ds(base, EPW)], dst_v)

    zeros = jnp.zeros((L,), jnp.float32)

    def zero_body(i, carry):
        hist_s[pl.ds(i * L, L)] = zeros
        hist_d[pl.ds(i * L, L)] = zeros
        return carry

    lax.fori_loop(0, N // L, zero_body, 0)

    ones = jnp.ones((L,), jnp.float32)

    def body(i, carry):
        si = src_v[pl.ds(i * L, L)]
        plsc.addupdate_scatter(hist_s, [si], ones)
        di = dst_v[pl.ds(i * L, L)]
        plsc.addupdate_scatter(hist_d, [di], ones)
        return carry

    lax.fori_loop(0, EPW // L, body, 0)

    pltpu.sync_copy(hist_s, out_hbm.at[0, w])
    pltpu.sync_copy(hist_d, out_hbm.at[1, w])


@functools.partial(
    pl.kernel,
    out_type=jax.ShapeDtypeStruct((NC, N, H), jnp.float32),
    mesh=_mesh,
    compiler_params=_sc_params,
    scratch_types=[
        pltpu.VMEM((CH,), jnp.int32),
        pltpu.VMEM((CH,), jnp.int32),
        pltpu.VMEM((CH, H), jnp.float32),
        pltpu.VMEM((RPT, H), jnp.float32),
        pltpu.VMEM_SHARED((N, H), jnp.float32),
        pltpu.SemaphoreType.DMA,
    ],
)
def _sc_agg_rows(g0_hbm, ei_hbm, out_hbm, idx_s, idx_d, rows, zbuf, acc_sp, sem):
    c = lax.axis_index("c")
    s = lax.axis_index("s")
    base = (s * NC + c) * EPW

    zeros = jnp.zeros((L,), jnp.float32)

    def zero_body(i, carry):
        zbuf[i, :] = zeros
        return carry

    lax.fori_loop(0, RPT, zero_body, 0)
    pltpu.sync_copy(zbuf, acc_sp.at[pl.ds(s * RPT, RPT)])
    plsc.subcore_barrier()

    def chunk(k, carry):
        off = base + k * CH
        pltpu.sync_copy(ei_hbm.at[0, pl.ds(off, CH)], idx_s)
        pltpu.sync_copy(ei_hbm.at[1, pl.ds(off, CH)], idx_d)
        pltpu.async_copy(g0_hbm.at[idx_s], rows, sem).wait()
        pltpu.sync_copy(rows, acc_sp.at[idx_d], add=True)
        return carry

    lax.fori_loop(0, EPW // CH, chunk, 0)
    plsc.subcore_barrier()

    pltpu.sync_copy(
        acc_sp.at[pl.ds(s * RPT, RPT)], out_hbm.at[c, pl.ds(s * RPT, RPT)]
    )


@functools.partial(
    pl.kernel,
    out_type=jax.ShapeDtypeStruct((NW, N), jnp.float32),
    mesh=_mesh,
    compiler_params=_sc_params,
    scratch_types=[
        pltpu.VMEM((N,), jnp.float32),
        pltpu.VMEM((EPW,), jnp.int32),
        pltpu.VMEM((EPW,), jnp.int32),
        pltpu.VMEM((N,), jnp.float32),
    ],
)
def _sc_agg_scalar(g1_hbm, ei_hbm, out_hbm, g1_v, src_v, dst_v, acc_v):
    w = _worker_id()
    base = w * EPW
    pltpu.sync_copy(g1_hbm, g1_v)
    pltpu.sync_copy(ei_hbm.at[0, pl.ds(base, EPW)], src_v)
    pltpu.sync_copy(ei_hbm.at[1, pl.ds(base, EPW)], dst_v)

    zeros = jnp.zeros((L,), jnp.float32)

    def zero_body(i, carry):
        acc_v[pl.ds(i * L, L)] = zeros
        return carry

    lax.fori_loop(0, N // L, zero_body, 0)

    def body(i, carry):
        iv = src_v[pl.ds(i * L, L)]
        vals = plsc.load_gather(g1_v, [iv])
        jv = dst_v[pl.ds(i * L, L)]
        plsc.addupdate_scatter(acc_v, [jv], vals)
        return carry

    lax.fori_loop(0, EPW // L, body, 0)

    pltpu.sync_copy(acc_v, out_hbm.at[w])


def _tc1_body(x_ref, w0_ref, degp_ref, g0_ref, dii_ref, dio_ref):
    deg_out = jnp.sum(degp_ref[0], axis=0) + 1.0
    deg_in = jnp.sum(degp_ref[1], axis=0) + 1.0
    dinv_out = lax.rsqrt(deg_out)
    dinv_in = lax.rsqrt(deg_in)
    h0 = jnp.dot(x_ref[...], w0_ref[...], preferred_element_type=jnp.float32)
    g0_ref[...] = h0 * dinv_out[:, None]
    dii_ref[...] = dinv_in
    dio_ref[...] = dinv_out


def _tc1(x, w0, degp):
    return pl.pallas_call(
        _tc1_body,
        out_shape=(
            jax.ShapeDtypeStruct((N, H), jnp.float32),
            jax.ShapeDtypeStruct((N,), jnp.float32),
            jax.ShapeDtypeStruct((N,), jnp.float32),
        ),
    )(x, w0, degp)


def _tc2_body(accp_ref, g0_ref, dii_ref, dio_ref, b0_ref, w1_ref, g1_ref):
    acc = accp_ref[0] + accp_ref[1] + g0_ref[...]
    h1 = jnp.maximum(acc * dii_ref[...][:, None] + b0_ref[...], 0.0)
    z = jnp.sum(h1 * w1_ref[...][:, 0][None, :], axis=1)
    g1_ref[...] = z * dio_ref[...]


def _tc2(accp, g0, dinv_in, dinv_out, b0, w1):
    return pl.pallas_call(
        _tc2_body,
        out_shape=jax.ShapeDtypeStruct((N,), jnp.float32),
    )(accp, g0, dinv_in, dinv_out, b0, w1)


def _tc3_body(accp_ref, g1_ref, dii_ref, b1_ref, out_ref):
    acc = jnp.sum(accp_ref[...], axis=0) + g1_ref[...]
    pre = acc * dii_ref[...] + b1_ref[0]
    out_ref[...] = jax.nn.sigmoid(pre)[:, None]


def _tc3(accp, g1, dinv_in, b1):
    return pl.pallas_call(
        _tc3_body,
        out_shape=jax.ShapeDtypeStruct((N, 1), jnp.float32),
    )(accp, g1, dinv_in, b1)


def kernel(x, edge_index, W0, b0, W1, b1):
    degp = _sc_degrees(edge_index)
    g0, dinv_in, dinv_out = _tc1(x, W0, degp)
    accp0 = _sc_agg_rows(g0, edge_index)
    g1 = _tc2(accp0, g0, dinv_in, dinv_out, b0, W1)
    accp1 = _sc_agg_scalar(g1, edge_index)
    return _tc3(accp1, g1, dinv_in, b1)


# speedup vs baseline: 73.5387x; 2.1432x over previous
import functools

import jax
import jax.numpy as jnp
from jax import lax
from jax.experimental import pallas as pl
from jax.experimental.pallas import tpu as pltpu
from jax.experimental.pallas import tpu_sc as plsc

N = 10000
E = 320000
D = 128
H = 16

NC = 2
NS = 16
L = 16
NW = NC * NS
EPW = E // NW
RPT = N // NS
CH = 80
NCH = EPW // CH
BLK = 5
NBLK = NCH // BLK

_mesh = plsc.VectorSubcoreMesh(
    core_axis_name="c", subcore_axis_name="s", num_cores=NC, num_subcores=NS
)
_sc_params = pltpu.CompilerParams(
    use_tc_tiling_on_sc=False, needs_layout_passes=False
)


def _worker_id():
    return lax.axis_index("s") * NC + lax.axis_index("c")


@functools.partial(
    pl.kernel,
    out_type=jax.ShapeDtypeStruct((2, NW, N), jnp.float32),
    mesh=_mesh,
    compiler_params=_sc_params,
    scratch_types=[
        pltpu.VMEM((EPW,), jnp.int32),
        pltpu.VMEM((EPW,), jnp.int32),
        pltpu.VMEM((N,), jnp.float32),
        pltpu.VMEM((N,), jnp.float32),
    ],
)
def _sc_degrees(ei_hbm, out_hbm, src_v, dst_v, hist_s, hist_d):
    w = _worker_id()
    base = w * EPW
    pltpu.sync_copy(ei_hbm.at[0, pl.ds(base, EPW)], src_v)
    pltpu.sync_copy(ei_hbm.at[1, pl.ds(base, EPW)], dst_v)

    zeros = jnp.zeros((L,), jnp.float32)

    def zero_body(i, carry):
        hist_s[pl.ds(i * L, L)] = zeros
        hist_d[pl.ds(i * L, L)] = zeros
        return carry

    lax.fori_loop(0, N // L, zero_body, 0)

    ones = jnp.ones((L,), jnp.float32)

    def body(i, carry):
        si = src_v[pl.ds(i * L, L)]
        plsc.addupdate_scatter(hist_s, [si], ones)
        di = dst_v[pl.ds(i * L, L)]
        plsc.addupdate_scatter(hist_d, [di], ones)
        return carry

    lax.fori_loop(0, EPW // L, body, 0)

    pltpu.sync_copy(hist_s, out_hbm.at[0, w])
    pltpu.sync_copy(hist_d, out_hbm.at[1, w])


@functools.partial(
    pl.kernel,
    out_type=jax.ShapeDtypeStruct((NC, N, H), jnp.float32),
    mesh=_mesh,
    compiler_params=_sc_params,
    scratch_types=[
        pltpu.VMEM((NCH, CH), jnp.int32),
        pltpu.VMEM((NCH, CH), jnp.int32),
        pltpu.VMEM((BLK * CH, H), jnp.float32),
        pltpu.VMEM((RPT, H), jnp.float32),
        pltpu.VMEM_SHARED((N, H), jnp.float32),
        pltpu.SemaphoreType.DMA,
        pltpu.SemaphoreType.DMA,
    ],
)
def _sc_agg_rows(
    g0_hbm, srcr_hbm, dstr_hbm, out_hbm, idx_sv, idx_dv, rows, zbuf, acc_sp,
    gsem, ssem,
):
    c = lax.axis_index("c")
    s = lax.axis_index("s")
    w = s * NC + c

    zeros = jnp.zeros((L,), jnp.float32)

    def zero_body(i, carry):
        zbuf[i, :] = zeros
        return carry

    lax.fori_loop(0, RPT, zero_body, 0)
    pltpu.sync_copy(zbuf, acc_sp.at[pl.ds(s * RPT, RPT)])
    pltpu.sync_copy(srcr_hbm.at[pl.ds(w * NCH, NCH)], idx_sv)
    pltpu.sync_copy(dstr_hbm.at[pl.ds(w * NCH, NCH)], idx_dv)
    plsc.subcore_barrier()

    def block(bk, carry):
        k0 = bk * BLK
        gds = [
            pltpu.async_copy(
                g0_hbm.at[idx_sv.at[k0 + j]],
                rows.at[pl.ds(j * CH, CH)],
                gsem,
            )
            for j in range(BLK)
        ]
        for d in gds:
            d.wait()
        sds = [
            pltpu.async_copy(
                rows.at[pl.ds(j * CH, CH)],
                acc_sp.at[idx_dv.at[k0 + j]],
                ssem,
                add=True,
            )
            for j in range(BLK)
        ]
        for d in sds:
            d.wait()
        return carry

    lax.fori_loop(0, NBLK, block, 0)
    plsc.subcore_barrier()

    pltpu.sync_copy(
        acc_sp.at[pl.ds(s * RPT, RPT)], out_hbm.at[c, pl.ds(s * RPT, RPT)]
    )


@functools.partial(
    pl.kernel,
    out_type=jax.ShapeDtypeStruct((NW, N), jnp.float32),
    mesh=_mesh,
    compiler_params=_sc_params,
    scratch_types=[
        pltpu.VMEM((N,), jnp.float32),
        pltpu.VMEM((EPW,), jnp.int32),
        pltpu.VMEM((EPW,), jnp.int32),
        pltpu.VMEM((N,), jnp.float32),
    ],
)
def _sc_agg_scalar(g1_hbm, ei_hbm, out_hbm, g1_v, src_v, dst_v, acc_v):
    w = _worker_id()
    base = w * EPW
    pltpu.sync_copy(g1_hbm, g1_v)
    pltpu.sync_copy(ei_hbm.at[0, pl.ds(base, EPW)], src_v)
    pltpu.sync_copy(ei_hbm.at[1, pl.ds(base, EPW)], dst_v)

    zeros = jnp.zeros((L,), jnp.float32)

    def zero_body(i, carry):
        acc_v[pl.ds(i * L, L)] = zeros
        return carry

    lax.fori_loop(0, N // L, zero_body, 0)

    def body(i, carry):
        iv = src_v[pl.ds(i * L, L)]
        vals = plsc.load_gather(g1_v, [iv])
        jv = dst_v[pl.ds(i * L, L)]
        plsc.addupdate_scatter(acc_v, [jv], vals)
        return carry

    lax.fori_loop(0, EPW // L, body, 0)

    pltpu.sync_copy(acc_v, out_hbm.at[w])


def _tc1_body(x_ref, w0_ref, degp_ref, g0_ref, dii_ref, dio_ref):
    deg_out = jnp.sum(degp_ref[0], axis=0) + 1.0
    deg_in = jnp.sum(degp_ref[1], axis=0) + 1.0
    dinv_out = lax.rsqrt(deg_out)
    dinv_in = lax.rsqrt(deg_in)
    h0 = jnp.dot(x_ref[...], w0_ref[...], preferred_element_type=jnp.float32)
    g0_ref[...] = h0 * dinv_out[:, None]
    dii_ref[...] = dinv_in
    dio_ref[...] = dinv_out


def _tc1(x, w0, degp):
    return pl.pallas_call(
        _tc1_body,
        out_shape=(
            jax.ShapeDtypeStruct((N, H), jnp.float32),
            jax.ShapeDtypeStruct((N,), jnp.float32),
            jax.ShapeDtypeStruct((N,), jnp.float32),
        ),
    )(x, w0, degp)


def _tc2_body(accp_ref, g0_ref, dii_ref, dio_ref, b0_ref, w1_ref, g1_ref):
    acc = accp_ref[0] + accp_ref[1] + g0_ref[...]
    h1 = jnp.maximum(acc * dii_ref[...][:, None] + b0_ref[...], 0.0)
    z = jnp.sum(h1 * w1_ref[...][:, 0][None, :], axis=1)
    g1_ref[...] = z * dio_ref[...]


def _tc2(accp, g0, dinv_in, dinv_out, b0, w1):
    return pl.pallas_call(
        _tc2_body,
        out_shape=jax.ShapeDtypeStruct((N,), jnp.float32),
    )(accp, g0, dinv_in, dinv_out, b0, w1)


def _tc3_body(accp_ref, g1_ref, dii_ref, b1_ref, out_ref):
    acc = jnp.sum(accp_ref[...], axis=0) + g1_ref[...]
    pre = acc * dii_ref[...] + b1_ref[0]
    out_ref[...] = jax.nn.sigmoid(pre)[:, None]


def _tc3(accp, g1, dinv_in, b1):
    return pl.pallas_call(
        _tc3_body,
        out_shape=jax.ShapeDtypeStruct((N, 1), jnp.float32),
    )(accp, g1, dinv_in, b1)


def kernel(x, edge_index, W0, b0, W1, b1):
    srcr = edge_index[0].reshape(NW * NCH, CH)
    dstr = edge_index[1].reshape(NW * NCH, CH)
    degp = _sc_degrees(edge_index)
    g0, dinv_in, dinv_out = _tc1(x, W0, degp)
    accp0 = _sc_agg_rows(g0, srcr, dstr)
    g1 = _tc2(accp0, g0, dinv_in, dinv_out, b0, W1)
    accp1 = _sc_agg_scalar(g1, edge_index)
    return _tc3(accp1, g1, dinv_in, b1)


# speedup vs baseline: 80.4757x; 1.0943x over previous
import functools

import jax
import jax.numpy as jnp
from jax import lax
from jax.experimental import pallas as pl
from jax.experimental.pallas import tpu as pltpu
from jax.experimental.pallas import tpu_sc as plsc

N = 10000
E = 320000
D = 128
H = 16

NC = 2
NS = 16
L = 16
NW = NC * NS
EPW = E // NW
RPT = N // NS
CH = 80
NCH = EPW // CH
RING = 5

_mesh = plsc.VectorSubcoreMesh(
    core_axis_name="c", subcore_axis_name="s", num_cores=NC, num_subcores=NS
)
_sc_params = pltpu.CompilerParams(
    use_tc_tiling_on_sc=False, needs_layout_passes=False
)


def _worker_id():
    return lax.axis_index("s") * NC + lax.axis_index("c")


@functools.partial(
    pl.kernel,
    out_type=jax.ShapeDtypeStruct((2, NW, N), jnp.float32),
    mesh=_mesh,
    compiler_params=_sc_params,
    scratch_types=[
        pltpu.VMEM((EPW,), jnp.int32),
        pltpu.VMEM((EPW,), jnp.int32),
        pltpu.VMEM((N,), jnp.float32),
        pltpu.VMEM((N,), jnp.float32),
    ],
)
def _sc_degrees(ei_hbm, out_hbm, src_v, dst_v, hist_s, hist_d):
    w = _worker_id()
    base = w * EPW
    pltpu.sync_copy(ei_hbm.at[0, pl.ds(base, EPW)], src_v)
    pltpu.sync_copy(ei_hbm.at[1, pl.ds(base, EPW)], dst_v)

    zeros = jnp.zeros((L,), jnp.float32)

    def zero_body(i, carry):
        hist_s[pl.ds(i * L, L)] = zeros
        hist_d[pl.ds(i * L, L)] = zeros
        return carry

    lax.fori_loop(0, N // L, zero_body, 0)

    ones = jnp.ones((L,), jnp.float32)

    def body(i, carry):
        si = src_v[pl.ds(i * L, L)]
        plsc.addupdate_scatter(hist_s, [si], ones)
        di = dst_v[pl.ds(i * L, L)]
        plsc.addupdate_scatter(hist_d, [di], ones)
        return carry

    lax.fori_loop(0, EPW // L, body, 0)

    pltpu.sync_copy(hist_s, out_hbm.at[0, w])
    pltpu.sync_copy(hist_d, out_hbm.at[1, w])


@functools.partial(
    pl.kernel,
    out_type=jax.ShapeDtypeStruct((NC, N, H), jnp.float32),
    mesh=_mesh,
    compiler_params=_sc_params,
    scratch_types=[
        pltpu.VMEM((NCH, CH), jnp.int32),
        pltpu.VMEM((NCH, CH), jnp.int32),
        pltpu.VMEM((RING, CH, H), jnp.float32),
        pltpu.VMEM((RPT, H), jnp.float32),
        pltpu.VMEM_SHARED((N, H), jnp.float32),
        pltpu.SemaphoreType.DMA((RING,)),
        pltpu.SemaphoreType.DMA((RING,)),
    ],
)
def _sc_agg_rows(
    g0_hbm, srcr_hbm, dstr_hbm, out_hbm, idx_sv, idx_dv, rows, zbuf, acc_sp,
    gsem, ssem,
):
    c = lax.axis_index("c")
    s = lax.axis_index("s")
    w = s * NC + c

    zeros = jnp.zeros((L,), jnp.float32)

    def zero_body(i, carry):
        zbuf[i, :] = zeros
        return carry

    lax.fori_loop(0, RPT, zero_body, 0)
    pltpu.sync_copy(zbuf, acc_sp.at[pl.ds(s * RPT, RPT)])
    pltpu.sync_copy(srcr_hbm.at[pl.ds(w * NCH, NCH)], idx_sv)
    pltpu.sync_copy(dstr_hbm.at[pl.ds(w * NCH, NCH)], idx_dv)
    plsc.subcore_barrier()

    def _wait_gather(k, j):
        pltpu.make_async_copy(
            g0_hbm.at[idx_sv.at[k]], rows.at[j], gsem.at[j]
        ).wait()

    def _scatter(k, j):
        pltpu.async_copy(
            rows.at[j], acc_sp.at[idx_dv.at[k]], ssem.at[j], add=True
        )

    def _wait_scatter(k, j):
        pltpu.make_async_copy(
            rows.at[j], acc_sp.at[idx_dv.at[k]], ssem.at[j]
        ).wait()

    for j in range(RING):
        pltpu.async_copy(g0_hbm.at[idx_sv.at[j]], rows.at[j], gsem.at[j])

    NIT = NCH // RING - 1

    def ring_body(i, carry):
        for j in range(RING):
            k = i * RING + j
            _wait_gather(k, j)
            _scatter(k, j)
            _wait_scatter(k, j)
            pltpu.async_copy(
                g0_hbm.at[idx_sv.at[k + RING]], rows.at[j], gsem.at[j]
            )
        return carry

    lax.fori_loop(0, NIT, ring_body, 0)

    for j in range(RING):
        k = NIT * RING + j
        _wait_gather(k, j)
        _scatter(k, j)
        _wait_scatter(k, j)
    plsc.subcore_barrier()

    pltpu.sync_copy(
        acc_sp.at[pl.ds(s * RPT, RPT)], out_hbm.at[c, pl.ds(s * RPT, RPT)]
    )


@functools.partial(
    pl.kernel,
    out_type=jax.ShapeDtypeStruct((NW, N), jnp.float32),
    mesh=_mesh,
    compiler_params=_sc_params,
    scratch_types=[
        pltpu.VMEM((N,), jnp.float32),
        pltpu.VMEM((EPW,), jnp.int32),
        pltpu.VMEM((EPW,), jnp.int32),
        pltpu.VMEM((N,), jnp.float32),
    ],
)
def _sc_agg_scalar(g1_hbm, ei_hbm, out_hbm, g1_v, src_v, dst_v, acc_v):
    w = _worker_id()
    base = w * EPW
    pltpu.sync_copy(g1_hbm, g1_v)
    pltpu.sync_copy(ei_hbm.at[0, pl.ds(base, EPW)], src_v)
    pltpu.sync_copy(ei_hbm.at[1, pl.ds(base, EPW)], dst_v)

    zeros = jnp.zeros((L,), jnp.float32)

    def zero_body(i, carry):
        acc_v[pl.ds(i * L, L)] = zeros
        return carry

    lax.fori_loop(0, N // L, zero_body, 0)

    def body(i, carry):
        iv = src_v[pl.ds(i * L, L)]
        vals = plsc.load_gather(g1_v, [iv])
        jv = dst_v[pl.ds(i * L, L)]
        plsc.addupdate_scatter(acc_v, [jv], vals)
        return carry

    lax.fori_loop(0, EPW // L, body, 0)

    pltpu.sync_copy(acc_v, out_hbm.at[w])


def _tc1_body(x_ref, w0_ref, degp_ref, g0_ref, dii_ref, dio_ref):
    deg_out = jnp.sum(degp_ref[0], axis=0) + 1.0
    deg_in = jnp.sum(degp_ref[1], axis=0) + 1.0
    dinv_out = lax.rsqrt(deg_out)
    dinv_in = lax.rsqrt(deg_in)
    h0 = jnp.dot(x_ref[...], w0_ref[...], preferred_element_type=jnp.float32)
    g0_ref[...] = h0 * dinv_out[:, None]
    dii_ref[...] = dinv_in
    dio_ref[...] = dinv_out


def _tc1(x, w0, degp):
    return pl.pallas_call(
        _tc1_body,
        out_shape=(
            jax.ShapeDtypeStruct((N, H), jnp.float32),
            jax.ShapeDtypeStruct((N,), jnp.float32),
            jax.ShapeDtypeStruct((N,), jnp.float32),
        ),
    )(x, w0, degp)


def _tc2_body(accp_ref, g0_ref, dii_ref, dio_ref, b0_ref, w1_ref, g1_ref):
    acc = accp_ref[0] + accp_ref[1] + g0_ref[...]
    h1 = jnp.maximum(acc * dii_ref[...][:, None] + b0_ref[...], 0.0)
    z = jnp.sum(h1 * w1_ref[...][:, 0][None, :], axis=1)
    g1_ref[...] = z * dio_ref[...]


def _tc2(accp, g0, dinv_in, dinv_out, b0, w1):
    return pl.pallas_call(
        _tc2_body,
        out_shape=jax.ShapeDtypeStruct((N,), jnp.float32),
    )(accp, g0, dinv_in, dinv_out, b0, w1)


def _tc3_body(accp_ref, g1_ref, dii_ref, b1_ref, out_ref):
    acc = jnp.sum(accp_ref[...], axis=0) + g1_ref[...]
    pre = acc * dii_ref[...] + b1_ref[0]
    out_ref[...] = jax.nn.sigmoid(pre)[:, None]


def _tc3(accp, g1, dinv_in, b1):
    return pl.pallas_call(
        _tc3_body,
        out_shape=jax.ShapeDtypeStruct((N, 1), jnp.float32),
    )(accp, g1, dinv_in, b1)


def kernel(x, edge_index, W0, b0, W1, b1):
    srcr = edge_index[0].reshape(NW * NCH, CH)
    dstr = edge_index[1].reshape(NW * NCH, CH)
    degp = _sc_degrees(edge_index)
    g0, dinv_in, dinv_out = _tc1(x, W0, degp)
    accp0 = _sc_agg_rows(g0, srcr, dstr)
    g1 = _tc2(accp0, g0, dinv_in, dinv_out, b0, W1)
    accp1 = _sc_agg_scalar(g1, edge_index)
    return _tc3(accp1, g1, dinv_in, b1)
